# Initial kernel scaffold; baseline (speedup 1.0000x reference)
import jax
import jax.numpy as jnp
from jax.experimental import pallas as pl


def kernel(inputs, targets):
    raise NotImplementedError("write your pallas kernel here")


# speedup vs baseline: 1.2719x; 1.2719x over previous
import functools

import jax
import jax.numpy as jnp
from jax.experimental import pallas as pl
from jax.experimental.pallas import tpu as pltpu

N = 4096
D = 128
ALPHA = 40.0
BETA = 2.0
MARGIN = 0.5

R = 256


def _body(xr_ref, xf_ref, tcol_ref, trow_ref, loss_ref, grad_ref):
    xr = xr_ref[...]
    xf = xf_ref[...]
    sim = jax.lax.dot_general(
        xr, xf, (((1,), (1,)), ((), ())),
        preferred_element_type=jnp.float32)

    tcol = tcol_ref[...]
    trow = trow_ref[...]
    same = tcol == trow
    pos_mask = same & (sim < 1.0)
    neg_mask = jnp.logical_not(same)

    inf = jnp.float32(jnp.inf)
    min_pos = jnp.min(jnp.where(pos_mask, sim, inf), axis=1, keepdims=True)
    max_neg = jnp.max(jnp.where(neg_mask, sim, -inf), axis=1, keepdims=True)

    neg_keep = neg_mask & (sim + 0.1 > min_pos)
    pos_keep = pos_mask & (sim - 0.1 < max_neg)
    valid = (jnp.any(pos_keep, axis=1, keepdims=True)
             & jnp.any(neg_keep, axis=1, keepdims=True))

    z_pos = -BETA * (sim - MARGIN)
    z_neg = ALPHA * (sim - MARGIN)
    sig_pos = jax.nn.sigmoid(z_pos)
    sig_neg = jax.nn.sigmoid(z_neg)
    pos_loss = (2.0 / BETA) * jnp.logaddexp(0.0, z_pos)
    neg_loss = (2.0 / ALPHA) * jnp.logaddexp(0.0, z_neg)

    one = jnp.float32(1.0)
    pos_cnt = jnp.maximum(
        jnp.sum(pos_keep.astype(jnp.float32), axis=1, keepdims=True), one)
    neg_cnt = jnp.maximum(
        jnp.sum(neg_keep.astype(jnp.float32), axis=1, keepdims=True), one)

    pos_sel = valid & pos_keep
    neg_sel = valid & neg_keep
    zero = jnp.float32(0.0)
    loss_ref[...] = (jnp.where(pos_sel, pos_loss, zero)
                     + jnp.where(neg_sel, neg_loss, zero))
    grad_ref[...] = (jnp.where(pos_sel, (-2.0 * sig_pos) / pos_cnt, zero)
                     + jnp.where(neg_sel, (2.0 * sig_neg) / neg_cnt, zero))


@jax.jit
def kernel(inputs, targets):
    tcol = targets.reshape(N, 1)
    trow = targets.reshape(1, N)
    grid = (N // R,)
    loss, grad = pl.pallas_call(
        _body,
        grid=grid,
        in_specs=[
            pl.BlockSpec((R, D), lambda i: (i, 0)),
            pl.BlockSpec((N, D), lambda i: (0, 0)),
            pl.BlockSpec((R, 1), lambda i: (i, 0)),
            pl.BlockSpec((1, N), lambda i: (0, 0)),
        ],
        out_specs=[
            pl.BlockSpec((R, N), lambda i: (i, 0)),
            pl.BlockSpec((R, N), lambda i: (i, 0)),
        ],
        out_shape=[
            jax.ShapeDtypeStruct((N, N), jnp.float32),
            jax.ShapeDtypeStruct((N, N), jnp.float32),
        ],
        compiler_params=pltpu.CompilerParams(
            dimension_semantics=("arbitrary",),
        ),
    )(inputs, inputs, tcol, trow)
    return loss.reshape(-1), grad.reshape(-1)


# speedup vs baseline: 1.6701x; 1.3131x over previous
import jax
import jax.numpy as jnp
from jax.experimental import pallas as pl
from jax.experimental.pallas import tpu as pltpu

N = 4096
D = 128
ALPHA = 40.0
BETA = 2.0
MARGIN = 0.5

R = 256


def _body(xr_ref, xf_ref, tcol_ref, trow_ref, loss_ref, grad_ref):
    xr = xr_ref[...]
    xf = xf_ref[...]
    sim = jax.lax.dot_general(
        xr, xf, (((1,), (1,)), ((), ())),
        preferred_element_type=jnp.float32)

    same = tcol_ref[...] == trow_ref[...]
    pos_mask = same & (sim < 1.0)

    inf = jnp.float32(jnp.inf)
    min_pos = jnp.min(jnp.where(pos_mask, sim, inf), axis=1, keepdims=True)
    max_neg = jnp.max(jnp.where(same, -inf, sim), axis=1, keepdims=True)

    neg_keep = jnp.logical_not(same) & (sim + 0.1 > min_pos)
    pos_keep = pos_mask & (sim - 0.1 < max_neg)

    pos_cnt_raw = jnp.sum(pos_keep.astype(jnp.float32), axis=1, keepdims=True)
    neg_cnt_raw = jnp.sum(neg_keep.astype(jnp.float32), axis=1, keepdims=True)
    valid = (pos_cnt_raw > 0.0) & (neg_cnt_raw > 0.0)

    one = jnp.float32(1.0)
    gp = -2.0 / jnp.maximum(pos_cnt_raw, one)
    gn = 2.0 / jnp.maximum(neg_cnt_raw, one)

    coeff = jnp.where(same, jnp.float32(-BETA), jnp.float32(ALPHA))
    z = coeff * (sim - MARGIN)
    az = jnp.abs(z)
    e = jnp.exp(-az)
    r = one / (one + e)
    lae = jnp.maximum(z, 0.0) + jnp.log1p(e)
    sig = jnp.where(z >= 0.0, r, e * r)

    loss_val = jnp.where(same, lae, (2.0 / ALPHA) * lae)
    grad_val = jnp.where(same, gp, gn) * sig

    keep = valid & (pos_keep | neg_keep)
    zero = jnp.float32(0.0)
    loss_ref[...] = jnp.where(keep, loss_val, zero)
    grad_ref[...] = jnp.where(keep, grad_val, zero)


@jax.jit
def kernel(inputs, targets):
    tcol = targets.reshape(N, 1)
    trow = targets.reshape(1, N)
    grid = (N // R,)
    loss, grad = pl.pallas_call(
        _body,
        grid=grid,
        in_specs=[
            pl.BlockSpec((R, D), lambda i: (i, 0)),
            pl.BlockSpec((N, D), lambda i: (0, 0)),
            pl.BlockSpec((R, 1), lambda i: (i, 0)),
            pl.BlockSpec((1, N), lambda i: (0, 0)),
        ],
        out_specs=[
            pl.BlockSpec((R, N), lambda i: (i, 0)),
            pl.BlockSpec((R, N), lambda i: (i, 0)),
        ],
        out_shape=[
            jax.ShapeDtypeStruct((N, N), jnp.float32),
            jax.ShapeDtypeStruct((N, N), jnp.float32),
        ],
        compiler_params=pltpu.CompilerParams(
            dimension_semantics=("arbitrary",),
        ),
    )(inputs, inputs, tcol, trow)
    return loss.reshape(-1), grad.reshape(-1)


# speedup vs baseline: 2.2874x; 1.3696x over previous
import jax
import jax.numpy as jnp
from jax.experimental import pallas as pl
from jax.experimental.pallas import tpu as pltpu

N = 4096
D = 128
ALPHA = 40.0
BETA = 2.0
MARGIN = 0.5

R = 256


def _body(xr_ref, xf_ref, tcol_ref, trow_ref, loss_ref, grad_ref):
    xr = xr_ref[...]
    xf = xf_ref[...]
    sim = jax.lax.dot_general(
        xr, xf, (((1,), (1,)), ((), ())),
        preferred_element_type=jnp.float32)

    same = tcol_ref[...] == trow_ref[...]
    pos_mask = same & (sim < 1.0)

    inf = jnp.float32(jnp.inf)
    min_pos = jnp.min(jnp.where(pos_mask, sim, inf), axis=1, keepdims=True)
    max_neg = jnp.max(jnp.where(same, -inf, sim), axis=1, keepdims=True)

    neg_keep = jnp.logical_not(same) & (sim + 0.1 > min_pos)
    pos_keep = pos_mask & (sim - 0.1 < max_neg)

    pos_cnt_raw = jnp.sum(pos_keep.astype(jnp.float32), axis=1, keepdims=True)
    neg_cnt_raw = jnp.sum(neg_keep.astype(jnp.float32), axis=1, keepdims=True)
    valid = (pos_cnt_raw > 0.0) & (neg_cnt_raw > 0.0)

    one = jnp.float32(1.0)
    gp = -2.0 / jnp.maximum(pos_cnt_raw, one)
    gn = 2.0 / jnp.maximum(neg_cnt_raw, one)

    coeff = jnp.where(same, jnp.float32(-BETA), jnp.float32(ALPHA))
    z = coeff * (sim - MARGIN)
    az = jnp.abs(z)
    e = jnp.exp(-az)
    r = one / (one + e)
    lae = jnp.maximum(z, 0.0) + jnp.log1p(e)
    sig = jnp.where(z >= 0.0, r, e * r)

    loss_val = jnp.where(same, lae, (2.0 / ALPHA) * lae)
    grad_val = jnp.where(same, gp, gn) * sig

    keep = valid & (pos_keep | neg_keep)
    zero = jnp.float32(0.0)
    loss_ref[...] = jnp.where(keep, loss_val, zero).reshape(R * N)
    grad_ref[...] = jnp.where(keep, grad_val, zero).reshape(R * N)


@jax.jit
def kernel(inputs, targets):
    tcol = targets.reshape(N, 1)
    trow = targets.reshape(1, N)
    grid = (N // R,)
    loss, grad = pl.pallas_call(
        _body,
        grid=grid,
        in_specs=[
            pl.BlockSpec((R, D), lambda i: (i, 0)),
            pl.BlockSpec((N, D), lambda i: (0, 0)),
            pl.BlockSpec((R, 1), lambda i: (i, 0)),
            pl.BlockSpec((1, N), lambda i: (0, 0)),
        ],
        out_specs=[
            pl.BlockSpec((R * N,), lambda i: (i,)),
            pl.BlockSpec((R * N,), lambda i: (i,)),
        ],
        out_shape=[
            jax.ShapeDtypeStruct((N * N,), jnp.float32),
            jax.ShapeDtypeStruct((N * N,), jnp.float32),
        ],
        compiler_params=pltpu.CompilerParams(
            dimension_semantics=("arbitrary",),
        ),
    )(inputs, inputs, tcol, trow)
    return loss, grad


# speedup vs baseline: 2.2977x; 1.0045x over previous
import jax
import jax.numpy as jnp
from jax.experimental import pallas as pl
from jax.experimental.pallas import tpu as pltpu

N = 4096
D = 128
ALPHA = 40.0
BETA = 2.0
MARGIN = 0.5

R = 256


def _body(xr_ref, xf_ref, tcol_ref, trow_ref, loss_ref, grad_ref):
    xr = xr_ref[...]
    xf = xf_ref[...]
    sim = jax.lax.dot_general(
        xr, xf, (((1,), (1,)), ((), ())),
        preferred_element_type=jnp.float32)

    same = tcol_ref[...] == trow_ref[...]
    pos_mask = same & (sim < 1.0)

    inf = jnp.float32(jnp.inf)
    min_pos = jnp.min(jnp.where(pos_mask, sim, inf), axis=1, keepdims=True)
    max_neg = jnp.max(jnp.where(same, -inf, sim), axis=1, keepdims=True)

    neg_keep = jnp.logical_not(same) & (sim + 0.1 > min_pos)
    pos_keep = pos_mask & (sim - 0.1 < max_neg)

    pos_cnt_raw = jnp.sum(pos_keep.astype(jnp.float32), axis=1, keepdims=True)
    neg_cnt_raw = jnp.sum(neg_keep.astype(jnp.float32), axis=1, keepdims=True)
    valid = (pos_cnt_raw > 0.0) & (neg_cnt_raw > 0.0)

    one = jnp.float32(1.0)
    gp = -2.0 / jnp.maximum(pos_cnt_raw, one)
    gn = 2.0 / jnp.maximum(neg_cnt_raw, one)

    coeff = jnp.where(same, jnp.float32(-BETA), jnp.float32(ALPHA))
    z = coeff * (sim - MARGIN)
    az = jnp.abs(z)
    e = jnp.exp(-az)
    r = one / (one + e)
    lae = jnp.maximum(z, 0.0) + jnp.log1p(e)
    sig = jnp.where(z >= 0.0, r, e * r)

    loss_val = jnp.where(same, lae, (2.0 / ALPHA) * lae)
    grad_val = jnp.where(same, gp, gn) * sig

    keep = valid & (pos_keep | neg_keep)
    zero = jnp.float32(0.0)
    loss_ref[...] = jnp.where(keep, loss_val, zero).reshape(R * N)
    grad_ref[...] = jnp.where(keep, grad_val, zero).reshape(R * N)


@jax.jit
def kernel(inputs, targets):
    tcol = targets.reshape(N, 1)
    trow = targets.reshape(1, N)
    grid = (N // R,)
    loss, grad = pl.pallas_call(
        _body,
        grid=grid,
        in_specs=[
            pl.BlockSpec((R, D), lambda i: (i, 0)),
            pl.BlockSpec((N, D), lambda i: (0, 0)),
            pl.BlockSpec((R, 1), lambda i: (i, 0)),
            pl.BlockSpec((1, N), lambda i: (0, 0)),
        ],
        out_specs=[
            pl.BlockSpec((R * N,), lambda i: (i,)),
            pl.BlockSpec((R * N,), lambda i: (i,)),
        ],
        out_shape=[
            jax.ShapeDtypeStruct((N * N,), jnp.float32),
            jax.ShapeDtypeStruct((N * N,), jnp.float32),
        ],
        compiler_params=pltpu.CompilerParams(
            dimension_semantics=("parallel",),
        ),
    )(inputs, inputs, tcol, trow)
    return loss, grad


# speedup vs baseline: 3.4464x; 1.4999x over previous
import jax
import jax.numpy as jnp
from jax.experimental import pallas as pl
from jax.experimental.pallas import tpu as pltpu

N = 4096
D = 128
ALPHA = 40.0
BETA = 2.0
MARGIN = 0.5

R = 256


def _body(xr_ref, xf_ref, tcol_ref, trow_ref, loss_ref, grad_ref):
    xr = xr_ref[...]
    xf = xf_ref[...]
    sim = jax.lax.dot_general(
        xr, xf, (((1,), (1,)), ((), ())),
        preferred_element_type=jnp.float32)

    same = tcol_ref[...] == trow_ref[...]

    inf = jnp.float32(jnp.inf)
    min_pos = jnp.min(
        jnp.where(same & (sim < 1.0), sim, inf), axis=1, keepdims=True)
    max_neg = jnp.max(jnp.where(same, -inf, sim), axis=1, keepdims=True)

    thr_p = jnp.minimum(jnp.float32(1.0), max_neg + 0.1)
    thr_n = min_pos - 0.1

    one = jnp.float32(1.0)
    zero = jnp.float32(0.0)
    c1 = jnp.where(sim < thr_p, one, zero)
    c2 = jnp.where(sim > thr_n, one, zero)
    keep_f = jnp.where(same, c1, c2)
    u_pos = jnp.where(same, keep_f, zero)
    ones_mat = jnp.ones((N, 128), jnp.float32)
    sums_all = jax.lax.dot_general(
        keep_f, ones_mat, (((1,), (0,)), ((), ())),
        preferred_element_type=jnp.float32)[:, 0:1]
    sums_pos = jax.lax.dot_general(
        u_pos, ones_mat, (((1,), (0,)), ((), ())),
        preferred_element_type=jnp.float32)[:, 0:1]
    pos_cnt = sums_pos
    neg_cnt = sums_all - sums_pos
    valid = (pos_cnt > zero) & (neg_cnt > zero)

    sp = jnp.where(valid, one, zero)
    sn = jnp.where(valid, jnp.float32(2.0 / ALPHA), zero)
    gp = jnp.where(valid, -2.0 / jnp.maximum(pos_cnt, one), zero)
    gn = jnp.where(valid, 2.0 / jnp.maximum(neg_cnt, one), zero)

    coeff = jnp.where(same, jnp.float32(-BETA), jnp.float32(ALPHA))
    z = coeff * (sim - MARGIN)
    az = jnp.abs(z)
    e = jnp.exp(-az)
    lae = jnp.maximum(z, zero) + jnp.log(one + e)
    sig = 0.5 + 0.5 * jnp.tanh(0.5 * z)

    loss_val = jnp.where(same, sp, sn) * lae
    grad_val = jnp.where(same, gp, gn) * sig

    loss_ref[...] = (keep_f * loss_val).reshape(R * N)
    grad_ref[...] = (keep_f * grad_val).reshape(R * N)


@jax.jit
def kernel(inputs, targets):
    tcol = targets.reshape(N, 1)
    trow = targets.reshape(1, N)
    grid = (N // R,)
    loss, grad = pl.pallas_call(
        _body,
        grid=grid,
        in_specs=[
            pl.BlockSpec((R, D), lambda i: (i, 0)),
            pl.BlockSpec((N, D), lambda i: (0, 0)),
            pl.BlockSpec((R, 1), lambda i: (i, 0)),
            pl.BlockSpec((1, N), lambda i: (0, 0)),
        ],
        out_specs=[
            pl.BlockSpec((R * N,), lambda i: (i,)),
            pl.BlockSpec((R * N,), lambda i: (i,)),
        ],
        out_shape=[
            jax.ShapeDtypeStruct((N * N,), jnp.float32),
            jax.ShapeDtypeStruct((N * N,), jnp.float32),
        ],
        compiler_params=pltpu.CompilerParams(
            dimension_semantics=("parallel",),
        ),
    )(inputs, inputs, tcol, trow)
    return loss, grad


# speedup vs baseline: 3.7668x; 1.0930x over previous
import jax
import jax.numpy as jnp
from jax.experimental import pallas as pl
from jax.experimental.pallas import tpu as pltpu

N = 4096
D = 128
ALPHA = 40.0
BETA = 2.0
MARGIN = 0.5

R = 256
NJ = N // 128


def _wait_step(step, loss_hbm, grad_hbm, ls_ref, gs_ref, sem_l, sem_g):
    slot = jax.lax.rem(step, 2)
    row0 = step * R
    for j in range(NJ):
        pltpu.make_async_copy(
            ls_ref.at[slot, j], loss_hbm.at[pl.ds(row0, R), j], sem_l).wait()
        pltpu.make_async_copy(
            gs_ref.at[slot, j], grad_hbm.at[pl.ds(row0, R), j], sem_g).wait()


def _body(xr_ref, xf_ref, tcol_ref, trow_ref, loss_hbm, grad_hbm,
          ls_ref, gs_ref, sem_l, sem_g):
    i = pl.program_id(0)
    nsteps = pl.num_programs(0)
    slot = jax.lax.rem(i, 2)

    @pl.when(i >= 2)
    def _():
        _wait_step(i - 2, loss_hbm, grad_hbm, ls_ref, gs_ref, sem_l, sem_g)

    xr = xr_ref[...]
    xf = xf_ref[...]
    sim = jax.lax.dot_general(
        xr, xf, (((1,), (1,)), ((), ())),
        preferred_element_type=jnp.float32)

    same = tcol_ref[...] == trow_ref[...]

    inf = jnp.float32(jnp.inf)
    min_pos = jnp.min(
        jnp.where(same & (sim < 1.0), sim, inf), axis=1, keepdims=True)
    max_neg = jnp.max(jnp.where(same, -inf, sim), axis=1, keepdims=True)

    thr_p = jnp.minimum(jnp.float32(1.0), max_neg + 0.1)
    thr_n = min_pos - 0.1

    one = jnp.float32(1.0)
    zero = jnp.float32(0.0)
    c1 = jnp.where(sim < thr_p, one, zero)
    c2 = jnp.where(sim > thr_n, one, zero)
    keep_f = jnp.where(same, c1, c2)
    u_pos = jnp.where(same, keep_f, zero)
    ones_mat = jnp.ones((N, 128), jnp.float32)
    sums_all = jax.lax.dot_general(
        keep_f, ones_mat, (((1,), (0,)), ((), ())),
        preferred_element_type=jnp.float32)[:, 0:1]
    sums_pos = jax.lax.dot_general(
        u_pos, ones_mat, (((1,), (0,)), ((), ())),
        preferred_element_type=jnp.float32)[:, 0:1]
    pos_cnt = sums_pos
    neg_cnt = sums_all - sums_pos
    valid = (pos_cnt > zero) & (neg_cnt > zero)

    sp = jnp.where(valid, one, zero)
    sn = jnp.where(valid, jnp.float32(2.0 / ALPHA), zero)
    gp = jnp.where(valid, -2.0 / jnp.maximum(pos_cnt, one), zero)
    gn = jnp.where(valid, 2.0 / jnp.maximum(neg_cnt, one), zero)

    coeff = jnp.where(same, jnp.float32(-BETA), jnp.float32(ALPHA))
    z = coeff * (sim - MARGIN)
    az = jnp.abs(z)
    e = jnp.exp(-az)
    lae = jnp.maximum(z, zero) + jnp.log(one + e)
    sig = 0.5 + 0.5 * jnp.tanh(0.5 * z)

    loss_val = keep_f * (jnp.where(same, sp, sn) * lae)
    grad_val = keep_f * (jnp.where(same, gp, gn) * sig)

    for j in range(NJ):
        ls_ref[slot, j] = loss_val[:, 128 * j:128 * (j + 1)]
        gs_ref[slot, j] = grad_val[:, 128 * j:128 * (j + 1)]

    row0 = i * R
    for j in range(NJ):
        pltpu.make_async_copy(
            ls_ref.at[slot, j], loss_hbm.at[pl.ds(row0, R), j], sem_l).start()
        pltpu.make_async_copy(
            gs_ref.at[slot, j], grad_hbm.at[pl.ds(row0, R), j], sem_g).start()

    @pl.when(i == nsteps - 1)
    def _():
        _wait_step(i - 1, loss_hbm, grad_hbm, ls_ref, gs_ref, sem_l, sem_g)
        _wait_step(i, loss_hbm, grad_hbm, ls_ref, gs_ref, sem_l, sem_g)


@jax.jit
def kernel(inputs, targets):
    tcol = targets.reshape(N, 1)
    trow = targets.reshape(1, N)
    grid = (N // R,)
    loss, grad = pl.pallas_call(
        _body,
        grid=grid,
        in_specs=[
            pl.BlockSpec((R, D), lambda i: (i, 0)),
            pl.BlockSpec((N, D), lambda i: (0, 0)),
            pl.BlockSpec((R, 1), lambda i: (i, 0)),
            pl.BlockSpec((1, N), lambda i: (0, 0)),
        ],
        out_specs=[
            pl.BlockSpec(memory_space=pltpu.MemorySpace.HBM),
            pl.BlockSpec(memory_space=pltpu.MemorySpace.HBM),
        ],
        out_shape=[
            jax.ShapeDtypeStruct((N, NJ, 128), jnp.float32),
            jax.ShapeDtypeStruct((N, NJ, 128), jnp.float32),
        ],
        scratch_shapes=[
            pltpu.VMEM((2, NJ, R, 128), jnp.float32),
            pltpu.VMEM((2, NJ, R, 128), jnp.float32),
            pltpu.SemaphoreType.DMA,
            pltpu.SemaphoreType.DMA,
        ],
        compiler_params=pltpu.CompilerParams(
            dimension_semantics=("arbitrary",),
        ),
    )(inputs, inputs, tcol, trow)
    return loss.reshape(-1), grad.reshape(-1)
